# baseline (speedup 1.0000x reference)
import functools

import jax
import jax.numpy as jnp
import numpy as np
from jax import lax
from jax.experimental import pallas as pl
from jax.experimental.pallas import tpu as pltpu
from jax.experimental.pallas import tpu_sc as plsc

N = 10000
E = 160000
E2 = E + N
HID = 512
EPS = 1e-5

_SPLITS = (np.cumsum(np.array([666] * 6 + [663] * 9, np.int64)) * 16).tolist()

NW = 32
VPW = 320
WIN = 64
NBLK = 64


def _agg_body(xw_hbm, ssrc_hbm, norm_hbm, ctl_hbm, sdst_hbm, woff_hbm, out_hbm,
              idx_v, norm_v, ctl_v, sdst_v, woff_v, rows_v, acc_v, pend_v,
              stag_v, sbase_s, sem, gsem):
    w = lax.axis_index("s") * 2 + lax.axis_index("c")
    v0 = w * VPW
    v1 = jnp.minimum(v0 + VPW, N)

    pltpu.sync_copy(woff_hbm, woff_v.at[pl.ds(0, 48)])

    def sget(ref, i):
        return ref[pl.ds(i, 16)][0]

    e0 = sget(woff_v, w)
    e1 = sget(woff_v, w + 1)

    zero16 = jnp.zeros((16,), jnp.float32)

    def zero_row(ref, r):
        for kk in range(HID // 16):
            ref[r, pl.ds(16 * kk, 16)] = zero16

    def zero_vec(ref):
        for kk in range(HID // 16):
            ref[pl.ds(16 * kk, 16)] = zero16

    zero_vec(acc_v)
    zero_vec(pend_v)

    @pl.loop(0, NBLK)
    def _(r):
        zero_row(stag_v, r)

    sbase_s[0] = v0

    def flush_block():
        sb = pl.multiple_of(sbase_s[0], NBLK)

        @pl.when(sb + NBLK <= N)
        def _():
            pltpu.sync_copy(stag_v, out_hbm.at[pl.ds(sb, NBLK)])

        @pl.when(sb + NBLK > N)
        def _():
            pltpu.sync_copy(stag_v.at[pl.ds(0, N % NBLK if N % NBLK else NBLK)],
                            out_hbm.at[pl.ds(sb, N % NBLK if N % NBLK else NBLK)])

        @pl.loop(0, NBLK)
        def _(r):
            zero_row(stag_v, r)

        sbase_s[0] = sb + NBLK

    k0 = e0 // WIN
    k1 = (e1 + WIN - 1) // WIN

    @pl.loop(0, (E2 + WIN - 1) // WIN)
    def _(k):
        @pl.when(jnp.logical_and(k >= k0, k < k1))
        def _():
            base = pl.multiple_of(k * WIN, WIN)
            pltpu.sync_copy(ssrc_hbm.at[pl.ds(base, WIN)], idx_v)
            pltpu.sync_copy(norm_hbm.at[pl.ds(base, WIN)], norm_v.at[pl.ds(0, WIN)])
            pltpu.sync_copy(ctl_hbm.at[pl.ds(base, WIN)], ctl_v.at[pl.ds(0, WIN)])
            pltpu.sync_copy(sdst_hbm.at[pl.ds(base, WIN)], sdst_v.at[pl.ds(0, WIN)])
            pltpu.async_copy(xw_hbm.at[idx_v], rows_v, gsem).wait()
            jlo = jnp.maximum(e0 - base, 0)
            jhi = jnp.minimum(e1 - base, WIN)

            @pl.loop(jlo, jhi)
            def _(j):
                nrm = sget(norm_v, j)
                for kk in range(HID // 16):
                    sl = pl.ds(16 * kk, 16)
                    plsc.addupdate(acc_v.at[sl], rows_v[j, sl] * nrm)
                cc = sget(ctl_v, j)

                @pl.when(cc == 1)
                def _():
                    for kk in range(HID // 16):
                        sl = pl.ds(16 * kk, 16)
                        pend_v[sl] = pend_v[sl] + acc_v[sl]
                        acc_v[sl] = zero16

                @pl.when(cc == 2)
                def _():
                    row = sget(sdst_v, j)
                    for _ in range(VPW // NBLK):
                        @pl.when(row >= sbase_s[0] + NBLK)
                        def _():
                            flush_block()
                    r = row - sbase_s[0]
                    for kk in range(HID // 16):
                        sl = pl.ds(16 * kk, 16)
                        stag_v[r, sl] = pend_v[sl] + acc_v[sl]
                        pend_v[sl] = zero16
                        acc_v[sl] = zero16

    for _ in range(VPW // NBLK):
        @pl.when(sbase_s[0] < v1)
        def _():
            flush_block()


@jax.jit
def _sc_aggregate(xw, ssrc, norm, ctl, sdst, woff):
    mesh = plsc.VectorSubcoreMesh(core_axis_name="c", subcore_axis_name="s")
    kern = pl.kernel(
        _agg_body,
        out_type=jax.ShapeDtypeStruct((N, HID), jnp.float32),
        mesh=mesh,
        scratch_types=[
            pltpu.VMEM((WIN,), jnp.int32),
            pltpu.VMEM((WIN + 16,), jnp.float32),
            pltpu.VMEM((WIN + 16,), jnp.int32),
            pltpu.VMEM((WIN + 16,), jnp.int32),
            pltpu.VMEM((64,), jnp.int32),
            pltpu.VMEM((WIN, HID), jnp.float32),
            pltpu.VMEM((HID,), jnp.float32),
            pltpu.VMEM((HID,), jnp.float32),
            pltpu.VMEM((NBLK, HID), jnp.float32),
            pltpu.SMEM((8,), jnp.int32),
            pltpu.SemaphoreType.DMA,
            pltpu.SemaphoreType.DMA,
        ],
    )
    return kern(xw, ssrc, norm, ctl, sdst, woff)


def _mm_body(a_ref, w_ref, o_ref):
    o_ref[...] = jnp.dot(a_ref[...], w_ref[...],
                         preferred_element_type=jnp.float32)


def _tc_matmul(a, w):
    br = 1000
    return pl.pallas_call(
        _mm_body,
        grid=(N // br,),
        in_specs=[pl.BlockSpec((br, a.shape[1]), lambda i: (i, 0)),
                  pl.BlockSpec(w.shape, lambda i: (0, 0))],
        out_specs=pl.BlockSpec((br, w.shape[1]), lambda i: (i, 0)),
        out_shape=jax.ShapeDtypeStruct((N, w.shape[1]), jnp.float32),
    )(a, w)


def _batch_norm(x, g, b):
    mean = jnp.mean(x, axis=0)
    var = jnp.mean((x - mean) ** 2, axis=0)
    return (x - mean) / jnp.sqrt(var + EPS) * g + b


def kernel(x, edge_index, W1, b1, W2, b2, W3, b3, g1, be1, g2, be2, g3, be3,
           Wc1, bc1, Wc2, bc2):
    loop = jnp.arange(N, dtype=edge_index.dtype)
    src = jnp.concatenate([edge_index[0], loop])
    dst = jnp.concatenate([edge_index[1], loop])

    order = jnp.argsort(dst, stable=True)
    ssrc = src[order]
    sdst = dst[order]
    off = jnp.searchsorted(sdst, jnp.arange(N + 1, dtype=jnp.int32))

    deg = (off[1:] - off[:-1]).astype(x.dtype)
    dinv = 1.0 / jnp.sqrt(jnp.clip(deg, 1.0))
    norm = dinv[ssrc] * dinv[sdst]

    is_last = jnp.concatenate([sdst[:-1] != sdst[1:],
                               jnp.ones((1,), bool)])
    splits = jnp.zeros((E2 + 1,), bool).at[jnp.array(_SPLITS)].set(True)[1:]
    ctl = jnp.where(is_last, 2, jnp.where(splits, 1, 0)).astype(jnp.int32)

    wnodes = jnp.minimum(jnp.arange(NW + 1, dtype=jnp.int32) * VPW, N)
    woff = jnp.pad(off[wnodes].astype(jnp.int32), (0, 48 - (NW + 1)))

    pad = (-E2) % WIN
    ssrc32 = jnp.pad(ssrc.astype(jnp.int32), (0, pad))
    sdst32 = jnp.pad(sdst.astype(jnp.int32), (0, pad))
    norm = jnp.pad(norm, (0, pad))
    ctl = jnp.pad(ctl, (0, pad))

    def gcn(h, W, b):
        xw = _tc_matmul(h, W)
        agg = _sc_aggregate(xw, ssrc32, norm, ctl, sdst32, woff)
        return agg + b

    h = gcn(x, W1, b1)
    h = _batch_norm(h, g1, be1)
    h = jax.nn.relu(h)
    h = gcn(h, W2, b2)
    h = _batch_norm(h, g2, be2)
    h = jax.nn.relu(h)
    h = gcn(h, W3, b3)
    h = _batch_norm(h, g3, be3)
    h = jnp.mean(h, axis=0, keepdims=True)
    h = jax.nn.relu(h @ Wc1 + bc1)
    return h @ Wc2 + bc2
